# baseline (speedup 1.0000x reference)
import functools

import jax
import jax.numpy as jnp
from jax import lax
from jax.experimental import pallas as pl
from jax.experimental.pallas import tpu as pltpu
from jax.experimental.pallas import tpu_sc as plsc

DIV = 3600
MOD = 168
D = 64

NC = 2
NS = 16
NW = NC * NS

CR = 2
GSUB = 80
NBUF = 2


def kernel(x, emb):
    n_rows, n_cols = x.shape
    K = CR * n_cols
    nsub = K // GSUB
    assert K % GSUB == 0 and GSUB % 16 == 0
    assert n_rows % (NW * CR * NBUF) == 0
    rows_per_w = n_rows // NW
    chunks = rows_per_w // CR

    B = n_rows * n_cols
    xf = x.reshape(B)
    mesh = plsc.VectorSubcoreMesh(core_axis_name="c", subcore_axis_name="s")

    @functools.partial(
        pl.kernel,
        mesh=mesh,
        out_type=jax.ShapeDtypeStruct((n_rows, n_cols, D), jnp.float32),
        scratch_types=[
            [pltpu.VMEM((K,), jnp.int32) for _ in range(NBUF)],
            [pltpu.VMEM((nsub, GSUB), jnp.int32) for _ in range(NBUF)],
            [pltpu.VMEM((K, D), jnp.float32) for _ in range(NBUF)],
            [pltpu.SemaphoreType.DMA for _ in range(NBUF)],
            [pltpu.SemaphoreType.DMA for _ in range(NBUF)],
            [pltpu.SemaphoreType.DMA for _ in range(NBUF)],
            pltpu.VMEM_SHARED((MOD, D), jnp.float32),
        ],
        compiler_params=pltpu.CompilerParams(use_tc_tiling_on_sc=False),
    )
    def k(x_hbm, emb_hbm, out_hbm, x_v, idx_v, rows_v, xsem, gsem, wsem,
          emb_sh):
        wid = lax.axis_index("s") * NC + lax.axis_index("c")
        w_row = wid * rows_per_w

        @pl.when(lax.axis_index("s") == 0)
        def _():
            pltpu.sync_copy(emb_hbm, emb_sh)
        plsc.subcore_barrier()

        def xload(c, b):
            return pltpu.make_async_copy(
                x_hbm.at[pl.ds((w_row + c * CR) * n_cols, K)], x_v[b],
                xsem[b])

        def gather(b, j):
            return pltpu.make_async_copy(
                emb_sh.at[idx_v[b].at[j]],
                rows_v[b].at[pl.ds(j * GSUB, GSUB)],
                gsem[b],
            )

        def wrout(c, b, r):
            return pltpu.make_async_copy(
                rows_v[b].at[pl.ds(r * n_cols, n_cols)],
                out_hbm.at[w_row + c * CR + r],
                wsem[b],
            )

        def compute_idx(b):
            for j in range(nsub):
                def vec_body(i, _, j=j):
                    xs = x_v[b][pl.ds(j * GSUB + i * 16, 16)]
                    q = (xs.astype(jnp.float32) * (1.0 / DIV)).astype(
                        jnp.int32)
                    r = xs - q * DIV
                    q = q + jnp.where(r >= DIV, 1, 0) - jnp.where(r < 0, 1, 0)
                    t = (q.astype(jnp.float32) * (1.0 / MOD)).astype(
                        jnp.int32)
                    m = q - t * MOD
                    m = jnp.where(m >= MOD, m - MOD, m)
                    m = jnp.where(m < 0, m + MOD, m)
                    idx_v[b][j, pl.ds(i * 16, 16)] = m
                    return 0
                lax.fori_loop(0, GSUB // 16, vec_body, 0)

        for b in range(NBUF):
            xload(b, b).start()

        def outer(i, _):
            for db in range(NBUF):
                c = i * NBUF + db
                prev = (db - 1) % NBUF
                xload(c, db).wait()
                compute_idx(db)

                @pl.when(c >= NBUF)
                def _():
                    for r in range(CR):
                        wrout(c - NBUF, db, r).wait()

                for j in range(nsub):
                    gather(db, j).start()

                @pl.when(c < chunks - NBUF)
                def _():
                    xload(c + NBUF, db).start()

                @pl.when(c >= 1)
                def _():
                    for j in range(nsub):
                        gather(prev, j).wait()
                    for r in range(CR):
                        wrout(c - 1, prev, r).start()
            return 0

        lax.fori_loop(0, chunks // NBUF, outer, 0)

        last = (chunks - 1) % NBUF
        for j in range(nsub):
            gather(last, j).wait()
        for r in range(CR):
            wrout(chunks - 1, last, r).start()
        for r in range(CR):
            wrout(chunks - NBUF, (last + 1) % NBUF, r).wait()
        for r in range(CR):
            wrout(chunks - 1, last, r).wait()

    return k(xf, emb)


# speedup vs baseline: 1.6904x; 1.6904x over previous
import functools

import jax
import jax.numpy as jnp
from jax import lax
from jax.experimental import pallas as pl
from jax.experimental.pallas import tpu as pltpu
from jax.experimental.pallas import tpu_sc as plsc

DIV = 3600
MOD = 168
D = 64
DP = 128

NC = 2
NS = 16
NW = NC * NS

GSUB = 128
NSUB = 1
K = GSUB * NSUB
NBUF = 4


def kernel(x, emb):
    n_rows, n_cols = x.shape
    B = n_rows * n_cols
    assert B % (NW * K * NBUF) == 0
    per_w = B // NW
    chunks = per_w // K

    xf = x.reshape(B)
    emb_p = jnp.pad(emb, ((0, 0), (0, DP - D)))
    mesh = plsc.VectorSubcoreMesh(core_axis_name="c", subcore_axis_name="s")

    @functools.partial(
        pl.kernel,
        mesh=mesh,
        out_type=jax.ShapeDtypeStruct((B, DP), jnp.float32),
        scratch_types=[
            [pltpu.VMEM((K,), jnp.int32) for _ in range(NBUF)],
            [pltpu.VMEM((NSUB, GSUB), jnp.int32) for _ in range(NBUF)],
            [pltpu.VMEM((K, DP), jnp.float32) for _ in range(NBUF)],
            [pltpu.SemaphoreType.DMA for _ in range(NBUF)],
            [pltpu.SemaphoreType.DMA for _ in range(NBUF)],
            [pltpu.SemaphoreType.DMA for _ in range(NBUF)],
            pltpu.VMEM_SHARED((MOD, DP), jnp.float32),
        ],
    )
    def k(x_hbm, emb_hbm, out_hbm, x_v, idx_v, rows_v, xsem, gsem, wsem,
          emb_sh):
        wid = lax.axis_index("s") * NC + lax.axis_index("c")
        w_base = wid * per_w

        @pl.when(lax.axis_index("s") == 0)
        def _():
            pltpu.sync_copy(emb_hbm, emb_sh)
        plsc.subcore_barrier()

        def xload(c, b):
            return pltpu.make_async_copy(
                x_hbm.at[pl.ds(w_base + c * K, K)], x_v[b], xsem[b])

        def gather(b, j):
            return pltpu.make_async_copy(
                emb_sh.at[idx_v[b].at[j]],
                rows_v[b].at[pl.ds(j * GSUB, GSUB)],
                gsem[b],
            )

        def wrout(c, b):
            return pltpu.make_async_copy(
                rows_v[b],
                out_hbm.at[pl.ds(w_base + c * K, K)],
                wsem[b],
            )

        def compute_idx(b):
            for j in range(NSUB):
                def vec_body(i, _, j=j):
                    xs = x_v[b][pl.ds(j * GSUB + i * 16, 16)]
                    q = (xs.astype(jnp.float32) * (1.0 / DIV)).astype(
                        jnp.int32)
                    r = xs - q * DIV
                    q = q + jnp.where(r >= DIV, 1, 0) - jnp.where(r < 0, 1, 0)
                    t = (q.astype(jnp.float32) * (1.0 / MOD)).astype(
                        jnp.int32)
                    m = q - t * MOD
                    m = jnp.where(m >= MOD, m - MOD, m)
                    m = jnp.where(m < 0, m + MOD, m)
                    idx_v[b][j, pl.ds(i * 16, 16)] = m
                    return 0
                lax.fori_loop(0, GSUB // 16, vec_body, 0)

        for b in range(NBUF):
            xload(b, b).start()

        def outer(i, _):
            for db in range(NBUF):
                c = i * NBUF + db
                prev = (db - 1) % NBUF
                xload(c, db).wait()
                compute_idx(db)

                @pl.when(c >= NBUF)
                def _():
                    wrout(c - NBUF, db).wait()

                for j in range(NSUB):
                    gather(db, j).start()

                @pl.when(c < chunks - NBUF)
                def _():
                    xload(c + NBUF, db).start()

                @pl.when(c >= 1)
                def _():
                    for j in range(NSUB):
                        gather(prev, j).wait()
                    wrout(c - 1, prev).start()
            return 0

        lax.fori_loop(0, chunks // NBUF, outer, 0)

        last = (chunks - 1) % NBUF
        for j in range(NSUB):
            gather(last, j).wait()
        wrout(chunks - 1, last).start()
        wrout(chunks - NBUF, (last + 1) % NBUF).wait()
        wrout(chunks - 1, last).wait()

    out = k(xf, emb_p)
    return out.reshape(n_rows, n_cols, DP)[:, :, :D]
